# baseline (speedup 1.0000x reference)
import functools

import jax
import jax.numpy as jnp
from jax import lax
from jax.experimental import pallas as pl
from jax.experimental.pallas import tpu as pltpu
from jax.experimental.pallas import tpu_sc as plsc


def _make_sc_gather(V2, D2, N):
    info = plsc.get_sparse_core_info()
    NC, NS = info.num_cores, info.num_subcores
    NW = NC * NS
    assert N % NW == 0
    n_per_w = N // NW
    CHUNK = 400
    assert n_per_w % CHUNK == 0
    n_chunks = n_per_w // CHUNK

    mesh = plsc.VectorSubcoreMesh(core_axis_name="c", subcore_axis_name="s")

    @functools.partial(
        pl.kernel,
        out_type=jax.ShapeDtypeStruct((N, D2), jnp.float32),
        mesh=mesh,
        scratch_types=[
            pltpu.VMEM((n_per_w,), jnp.int32),
            pltpu.VMEM((CHUNK, D2), jnp.float32),
            pltpu.VMEM((CHUNK, D2), jnp.float32),
            pltpu.SemaphoreType.DMA,
            pltpu.SemaphoreType.DMA,
        ],
    )
    def gather(table_hbm, idx_hbm, out_hbm, idx_v, rows_a, rows_b, sem_a, sem_b):
        wid = lax.axis_index("s") * NC + lax.axis_index("c")
        base = wid * n_per_w
        pltpu.sync_copy(idx_hbm.at[pl.ds(base, n_per_w)], idx_v)
        bufs = ((rows_a, sem_a), (rows_b, sem_b))
        copies = []
        for c in range(n_chunks):
            rows_v, sem = bufs[c % 2]
            if c >= 2:
                copies[c - 2].wait()
                pltpu.sync_copy(rows_v, out_hbm.at[pl.ds(base + (c - 2) * CHUNK, CHUNK)])
            copies.append(pltpu.async_copy(
                table_hbm.at[idx_v.at[pl.ds(c * CHUNK, CHUNK)]], rows_v, sem))
        for c in range(n_chunks - 2, n_chunks):
            rows_v, sem = bufs[c % 2]
            copies[c].wait()
            pltpu.sync_copy(rows_v, out_hbm.at[pl.ds(base + c * CHUNK, CHUNK)])

    return gather


def _lstm_body(H, L, xf_ref, xb_ref, pf_ref, pb_ref,
               wih_f, whh_f, b_f, wih_b, whh_b, b_b,
               hsf_ref, hsb_ref, hf, cf, hb, cb):
    t = pl.program_id(0)

    @pl.when(t == 0)
    def _():
        hf[...] = jnp.zeros_like(hf)
        cf[...] = jnp.zeros_like(cf)
        hb[...] = jnp.zeros_like(hb)
        cb[...] = jnp.zeros_like(cb)

    def step(xw, par, w_ih, w_hh, b, h_s, c_s, out_ref):
        x = jnp.where(par > 0.5, xw[:, H:2 * H], xw[:, 0:H])
        gates = (
            jnp.dot(x, w_ih[...], preferred_element_type=jnp.float32)
            + jnp.dot(h_s[...], w_hh[...], preferred_element_type=jnp.float32)
            + b[...]
        )
        i = jax.nn.sigmoid(gates[:, 0 * H:1 * H])
        f = jax.nn.sigmoid(gates[:, 1 * H:2 * H])
        g = jnp.tanh(gates[:, 2 * H:3 * H])
        o = jax.nn.sigmoid(gates[:, 3 * H:4 * H])
        c = f * c_s[...] + i * g
        h = o * jnp.tanh(c)
        c_s[...] = c
        h_s[...] = h
        out_ref[0] = h

    step(xf_ref[0], pf_ref[0], wih_f, whh_f, b_f, hf, cf, hsf_ref)
    step(xb_ref[0], pb_ref[0], wih_b, whh_b, b_b, hb, cb, hsb_ref)


def _run_lstm(xw, par, wih_f_T, whh_f_T, b_f, wih_b_T, whh_b_T, b_b,
              interpret=False):
    L, B, E2 = xw.shape
    H = whh_f_T.shape[0]
    const = lambda shape: pl.BlockSpec(shape, lambda t: (0,) * len(shape))
    return pl.pallas_call(
        functools.partial(_lstm_body, H, L),
        grid=(L,),
        in_specs=[
            pl.BlockSpec((1, B, E2), lambda t: (t, 0, 0)),
            pl.BlockSpec((1, B, E2), lambda t: (L - 1 - t, 0, 0)),
            pl.BlockSpec((1, B, 1), lambda t: (t, 0, 0)),
            pl.BlockSpec((1, B, 1), lambda t: (L - 1 - t, 0, 0)),
            const((H, 4 * H)), const((H, 4 * H)), const((1, 4 * H)),
            const((H, 4 * H)), const((H, 4 * H)), const((1, 4 * H)),
        ],
        out_specs=[
            pl.BlockSpec((1, B, H), lambda t: (t, 0, 0)),
            pl.BlockSpec((1, B, H), lambda t: (L - 1 - t, 0, 0)),
        ],
        out_shape=[
            jax.ShapeDtypeStruct((L, B, H), jnp.float32),
            jax.ShapeDtypeStruct((L, B, H), jnp.float32),
        ],
        scratch_shapes=[pltpu.VMEM((B, H), jnp.float32)] * 4,
        compiler_params=pltpu.CompilerParams(
            dimension_semantics=("arbitrary",),
        ),
        interpret=interpret,
    )(xw, xw, par, par, wih_f_T, whh_f_T, b_f, wih_b_T, whh_b_T, b_b)


def _attn_body(L, hsf_ref, hsb_ref, hnf_ref, hnb_ref, wof, wob, bo,
               out_ref, m, d, accf, accb):
    t = pl.program_id(0)

    @pl.when(t == 0)
    def _():
        m[...] = jnp.full_like(m, -jnp.inf)
        d[...] = jnp.zeros_like(d)
        accf[...] = jnp.zeros_like(accf)
        accb[...] = jnp.zeros_like(accb)

    hf = hsf_ref[0]
    hb = hsb_ref[0]
    s = (jnp.sum(hf * hnf_ref[0], axis=-1, keepdims=True)
         + jnp.sum(hb * hnb_ref[0], axis=-1, keepdims=True))
    m_new = jnp.maximum(m[...], s)
    alpha = jnp.exp(m[...] - m_new)
    p = jnp.exp(s - m_new)
    d[...] = d[...] * alpha + p
    accf[...] = accf[...] * alpha + p * hf
    accb[...] = accb[...] * alpha + p * hb
    m[...] = m_new

    @pl.when(t == L - 1)
    def _():
        inv = 1.0 / d[...]
        ctxf = accf[...] * inv
        ctxb = accb[...] * inv
        out_ref[...] = (
            jnp.dot(ctxf, wof[...], preferred_element_type=jnp.float32)
            + jnp.dot(ctxb, wob[...], preferred_element_type=jnp.float32)
            + bo[...]
        )


def _run_attn(hs_f, hs_b, wof, wob, bo, interpret=False):
    L, B, H = hs_f.shape
    C = wof.shape[1]
    const = lambda shape: pl.BlockSpec(shape, lambda t: (0,) * len(shape))
    return pl.pallas_call(
        functools.partial(_attn_body, L),
        grid=(L,),
        in_specs=[
            pl.BlockSpec((1, B, H), lambda t: (t, 0, 0)),
            pl.BlockSpec((1, B, H), lambda t: (t, 0, 0)),
            pl.BlockSpec((1, B, H), lambda t: (0, 0, 0)),
            pl.BlockSpec((1, B, H), lambda t: (L - 1, 0, 0)),
            const((H, C)), const((H, C)), const((1, C)),
        ],
        out_specs=pl.BlockSpec((B, C), lambda t: (0, 0)),
        out_shape=jax.ShapeDtypeStruct((B, C), jnp.float32),
        scratch_shapes=[
            pltpu.VMEM((B, 1), jnp.float32),
            pltpu.VMEM((B, 1), jnp.float32),
            pltpu.VMEM((B, H), jnp.float32),
            pltpu.VMEM((B, H), jnp.float32),
        ],
        compiler_params=pltpu.CompilerParams(
            dimension_semantics=("arbitrary",),
        ),
        interpret=interpret,
    )(hs_f, hs_b, hs_b, hs_f, wof, wob, bo)


def kernel(batch, emb, Wih_f, Whh_f, bih_f, bhh_f, Wih_b, Whh_b, bih_b, bhh_b,
           W_out, b_out):
    B, L = batch.shape
    V, E = emb.shape
    H = Whh_f.shape[1]
    C = W_out.shape[0]

    idx = batch.astype(jnp.int32).T.reshape(-1)
    emb2 = emb.reshape(V // 2, 2 * E)
    x_wide = _make_sc_gather(V // 2, 2 * E, L * B)(emb2, idx // 2)
    xw = x_wide.reshape(L, B, 2 * E)
    par = (idx & 1).astype(jnp.float32).reshape(L, B, 1)

    b_f = (bih_f + bhh_f).reshape(1, 4 * H)
    b_b = (bih_b + bhh_b).reshape(1, 4 * H)
    hs_f, hs_b = _run_lstm(xw, par, Wih_f.T, Whh_f.T, b_f, Wih_b.T, Whh_b.T, b_b)

    woutT = W_out.T
    out = _run_attn(hs_f, hs_b, woutT[:H], woutT[H:], b_out.reshape(1, C))
    return out


# speedup vs baseline: 1.2074x; 1.2074x over previous
import functools

import jax
import jax.numpy as jnp
from jax import lax
from jax.experimental import pallas as pl
from jax.experimental.pallas import tpu as pltpu
from jax.experimental.pallas import tpu_sc as plsc


def _make_sc_gather(V2, D2, N):
    info = plsc.get_sparse_core_info()
    NC, NS = info.num_cores, info.num_subcores
    NW = NC * NS
    assert N % NW == 0
    n_per_w = N // NW
    CHUNK = 400
    assert n_per_w % CHUNK == 0
    n_chunks = n_per_w // CHUNK

    mesh = plsc.VectorSubcoreMesh(core_axis_name="c", subcore_axis_name="s")

    @functools.partial(
        pl.kernel,
        out_type=jax.ShapeDtypeStruct((N, D2), jnp.float32),
        mesh=mesh,
        scratch_types=[
            pltpu.VMEM((n_per_w,), jnp.int32),
            pltpu.VMEM((CHUNK, D2), jnp.float32),
            pltpu.VMEM((CHUNK, D2), jnp.float32),
            pltpu.SemaphoreType.DMA,
            pltpu.SemaphoreType.DMA,
        ],
    )
    def gather(table_hbm, idx_hbm, out_hbm, idx_v, rows_a, rows_b, sem_a, sem_b):
        wid = lax.axis_index("s") * NC + lax.axis_index("c")
        base = wid * n_per_w
        pltpu.sync_copy(idx_hbm.at[pl.ds(base, n_per_w)], idx_v)
        bufs = ((rows_a, sem_a), (rows_b, sem_b))
        copies = []
        for c in range(n_chunks):
            rows_v, sem = bufs[c % 2]
            if c >= 2:
                copies[c - 2].wait()
                pltpu.sync_copy(rows_v, out_hbm.at[pl.ds(base + (c - 2) * CHUNK, CHUNK)])
            copies.append(pltpu.async_copy(
                table_hbm.at[idx_v.at[pl.ds(c * CHUNK, CHUNK)]], rows_v, sem))
        for c in range(n_chunks - 2, n_chunks):
            rows_v, sem = bufs[c % 2]
            copies[c].wait()
            pltpu.sync_copy(rows_v, out_hbm.at[pl.ds(base + c * CHUNK, CHUNK)])

    return gather


def _lstm_body(H, L, xf_ref, xb_ref,
               wih_f, whh_f, b_f, wih_b, whh_b, b_b,
               hsf_ref, hsb_ref, hf, cf, hb, cb):
    t = pl.program_id(0)

    @pl.when(t == 0)
    def _():
        hf[...] = jnp.zeros_like(hf)
        cf[...] = jnp.zeros_like(cf)
        hb[...] = jnp.zeros_like(hb)
        cb[...] = jnp.zeros_like(cb)

    def step(xw, w_ih, w_hh, b, h_s, c_s, out_ref):
        x = xw[:, 0:H]
        gates = (
            jnp.dot(x, w_ih[...], preferred_element_type=jnp.float32)
            + jnp.dot(h_s[...], w_hh[...], preferred_element_type=jnp.float32)
            + b[...]
        )
        i = jax.nn.sigmoid(gates[:, 0 * H:1 * H])
        f = jax.nn.sigmoid(gates[:, 1 * H:2 * H])
        g = jnp.tanh(gates[:, 2 * H:3 * H])
        o = jax.nn.sigmoid(gates[:, 3 * H:4 * H])
        c = f * c_s[...] + i * g
        h = o * jnp.tanh(c)
        c_s[...] = c
        h_s[...] = h
        out_ref[0] = h

    step(xf_ref[0], wih_f, whh_f, b_f, hf, cf, hsf_ref)
    step(xb_ref[0], wih_b, whh_b, b_b, hb, cb, hsb_ref)


def _run_lstm(xw, wih_f_T, whh_f_T, b_f, wih_b_T, whh_b_T, b_b,
              interpret=False):
    L, B, E2 = xw.shape
    H = whh_f_T.shape[0]
    const = lambda shape: pl.BlockSpec(shape, lambda t: (0,) * len(shape))
    return pl.pallas_call(
        functools.partial(_lstm_body, H, L),
        grid=(L,),
        in_specs=[
            pl.BlockSpec((1, B, E2), lambda t: (t, 0, 0)),
            pl.BlockSpec((1, B, E2), lambda t: (L - 1 - t, 0, 0)),
            const((H, 4 * H)), const((H, 4 * H)), const((1, 4 * H)),
            const((H, 4 * H)), const((H, 4 * H)), const((1, 4 * H)),
        ],
        out_specs=[
            pl.BlockSpec((1, B, H), lambda t: (t, 0, 0)),
            pl.BlockSpec((1, B, H), lambda t: (L - 1 - t, 0, 0)),
        ],
        out_shape=[
            jax.ShapeDtypeStruct((L, B, H), jnp.float32),
            jax.ShapeDtypeStruct((L, B, H), jnp.float32),
        ],
        scratch_shapes=[pltpu.VMEM((B, H), jnp.float32)] * 4,
        compiler_params=pltpu.CompilerParams(
            dimension_semantics=("arbitrary",),
        ),
        interpret=interpret,
    )(xw, xw, wih_f_T, whh_f_T, b_f, wih_b_T, whh_b_T, b_b)


def _attn_body(L, hsf_ref, hsb_ref, hnf_ref, hnb_ref, wof, wob, bo,
               out_ref, m, d, accf, accb):
    t = pl.program_id(0)

    @pl.when(t == 0)
    def _():
        m[...] = jnp.full_like(m, -jnp.inf)
        d[...] = jnp.zeros_like(d)
        accf[...] = jnp.zeros_like(accf)
        accb[...] = jnp.zeros_like(accb)

    hf = hsf_ref[0]
    hb = hsb_ref[0]
    s = (jnp.sum(hf * hnf_ref[0], axis=-1, keepdims=True)
         + jnp.sum(hb * hnb_ref[0], axis=-1, keepdims=True))
    m_new = jnp.maximum(m[...], s)
    alpha = jnp.exp(m[...] - m_new)
    p = jnp.exp(s - m_new)
    d[...] = d[...] * alpha + p
    accf[...] = accf[...] * alpha + p * hf
    accb[...] = accb[...] * alpha + p * hb
    m[...] = m_new

    @pl.when(t == L - 1)
    def _():
        inv = 1.0 / d[...]
        ctxf = accf[...] * inv
        ctxb = accb[...] * inv
        out_ref[...] = (
            jnp.dot(ctxf, wof[...], preferred_element_type=jnp.float32)
            + jnp.dot(ctxb, wob[...], preferred_element_type=jnp.float32)
            + bo[...]
        )


def _run_attn(hs_f, hs_b, wof, wob, bo, interpret=False):
    L, B, H = hs_f.shape
    C = wof.shape[1]
    const = lambda shape: pl.BlockSpec(shape, lambda t: (0,) * len(shape))
    return pl.pallas_call(
        functools.partial(_attn_body, L),
        grid=(L,),
        in_specs=[
            pl.BlockSpec((1, B, H), lambda t: (t, 0, 0)),
            pl.BlockSpec((1, B, H), lambda t: (t, 0, 0)),
            pl.BlockSpec((1, B, H), lambda t: (0, 0, 0)),
            pl.BlockSpec((1, B, H), lambda t: (L - 1, 0, 0)),
            const((H, C)), const((H, C)), const((1, C)),
        ],
        out_specs=pl.BlockSpec((B, C), lambda t: (0, 0)),
        out_shape=jax.ShapeDtypeStruct((B, C), jnp.float32),
        scratch_shapes=[
            pltpu.VMEM((B, 1), jnp.float32),
            pltpu.VMEM((B, 1), jnp.float32),
            pltpu.VMEM((B, H), jnp.float32),
            pltpu.VMEM((B, H), jnp.float32),
        ],
        compiler_params=pltpu.CompilerParams(
            dimension_semantics=("arbitrary",),
        ),
        interpret=interpret,
    )(hs_f, hs_b, hs_b, hs_f, wof, wob, bo)


def kernel(batch, emb, Wih_f, Whh_f, bih_f, bhh_f, Wih_b, Whh_b, bih_b, bhh_b,
           W_out, b_out):
    B, L = batch.shape
    V, E = emb.shape
    H = Whh_f.shape[1]
    C = W_out.shape[0]

    idx = batch.astype(jnp.int32).T.reshape(-1)
    emb2 = jnp.pad(emb, ((0, 0), (0, E)))
    x_wide = _make_sc_gather(V, 2 * E, L * B)(emb2, idx)
    xw = x_wide.reshape(L, B, 2 * E)

    b_f = (bih_f + bhh_f).reshape(1, 4 * H)
    b_b = (bih_b + bhh_b).reshape(1, 4 * H)
    hs_f, hs_b = _run_lstm(xw, Wih_f.T, Whh_f.T, b_f, Wih_b.T, Whh_b.T, b_b)

    woutT = W_out.T
    out = _run_attn(hs_f, hs_b, woutT[:H], woutT[H:], b_out.reshape(1, C))
    return out


# speedup vs baseline: 1.4401x; 1.1928x over previous
import functools

import jax
import jax.numpy as jnp
from jax import lax
from jax.experimental import pallas as pl
from jax.experimental.pallas import tpu as pltpu
from jax.experimental.pallas import tpu_sc as plsc


def _make_sc_gather(V2, D2, N):
    info = plsc.get_sparse_core_info()
    NC, NS = info.num_cores, info.num_subcores
    NW = NC * NS
    assert N % NW == 0
    n_per_w = N // NW
    CHUNK = 400
    assert n_per_w % CHUNK == 0
    n_chunks = n_per_w // CHUNK

    mesh = plsc.VectorSubcoreMesh(core_axis_name="c", subcore_axis_name="s")

    @functools.partial(
        pl.kernel,
        out_type=jax.ShapeDtypeStruct((N, D2), jnp.float32),
        mesh=mesh,
        scratch_types=[
            pltpu.VMEM((n_per_w,), jnp.int32),
            pltpu.VMEM((CHUNK, D2), jnp.float32),
            pltpu.VMEM((CHUNK, D2), jnp.float32),
            pltpu.SemaphoreType.DMA,
            pltpu.SemaphoreType.DMA,
        ],
    )
    def gather(table_hbm, idx_hbm, out_hbm, idx_v, rows_a, rows_b, sem_a, sem_b):
        wid = lax.axis_index("s") * NC + lax.axis_index("c")
        base = wid * n_per_w
        pltpu.sync_copy(idx_hbm.at[pl.ds(base, n_per_w)], idx_v)
        bufs = ((rows_a, sem_a), (rows_b, sem_b))
        copies = []
        for c in range(n_chunks):
            rows_v, sem = bufs[c % 2]
            if c >= 2:
                copies[c - 2].wait()
                pltpu.sync_copy(rows_v, out_hbm.at[pl.ds(base + (c - 2) * CHUNK, CHUNK)])
            copies.append(pltpu.async_copy(
                table_hbm.at[idx_v.at[pl.ds(c * CHUNK, CHUNK)]], rows_v, sem))
        for c in range(n_chunks - 2, n_chunks):
            rows_v, sem = bufs[c % 2]
            copies[c].wait()
            pltpu.sync_copy(rows_v, out_hbm.at[pl.ds(base + c * CHUNK, CHUNK)])

    return gather


def _pad_transpose_body(embT_ref, w_ref, out_ref):
    out_ref[...] = jax.lax.dot_general(
        embT_ref[...], w_ref[...], (((0,), (0,)), ((), ())),
        preferred_element_type=jnp.float32)


def _run_pad_transpose(embT):
    E, V = embT.shape
    CB = 4096
    nb = (V + CB - 1) // CB
    w = jnp.concatenate([jnp.eye(E, dtype=jnp.float32),
                         jnp.zeros((E, E), jnp.float32)], axis=1)
    return pl.pallas_call(
        _pad_transpose_body,
        grid=(nb,),
        in_specs=[
            pl.BlockSpec((E, CB), lambda j: (0, j)),
            pl.BlockSpec((E, 2 * E), lambda j: (0, 0)),
        ],
        out_specs=pl.BlockSpec((CB, 2 * E), lambda j: (j, 0)),
        out_shape=jax.ShapeDtypeStruct((V, 2 * E), jnp.float32),
        compiler_params=pltpu.CompilerParams(
            dimension_semantics=("arbitrary",),
        ),
    )(embT, w)


def _lstm_body(H, L, xf_ref, xb_ref,
               wih_f, whh_f, b_f, wih_b, whh_b, b_b,
               hsf_ref, hsb_ref, hf, cf, hb, cb):
    t = pl.program_id(0)

    @pl.when(t == 0)
    def _():
        hf[...] = jnp.zeros_like(hf)
        cf[...] = jnp.zeros_like(cf)
        hb[...] = jnp.zeros_like(hb)
        cb[...] = jnp.zeros_like(cb)

    def step(xw, w_ih, w_hh, b, h_s, c_s, out_ref):
        x = xw[:, 0:H]
        gates = (
            jnp.dot(x, w_ih[...], preferred_element_type=jnp.float32)
            + jnp.dot(h_s[...], w_hh[...], preferred_element_type=jnp.float32)
            + b[...]
        )
        i = jax.nn.sigmoid(gates[:, 0 * H:1 * H])
        f = jax.nn.sigmoid(gates[:, 1 * H:2 * H])
        g = jnp.tanh(gates[:, 2 * H:3 * H])
        o = jax.nn.sigmoid(gates[:, 3 * H:4 * H])
        c = f * c_s[...] + i * g
        h = o * jnp.tanh(c)
        c_s[...] = c
        h_s[...] = h
        out_ref[0] = h

    step(xf_ref[0], wih_f, whh_f, b_f, hf, cf, hsf_ref)
    step(xb_ref[0], wih_b, whh_b, b_b, hb, cb, hsb_ref)


def _run_lstm(xw, wih_f_T, whh_f_T, b_f, wih_b_T, whh_b_T, b_b,
              interpret=False):
    L, B, E2 = xw.shape
    H = whh_f_T.shape[0]
    const = lambda shape: pl.BlockSpec(shape, lambda t: (0,) * len(shape))
    return pl.pallas_call(
        functools.partial(_lstm_body, H, L),
        grid=(L,),
        in_specs=[
            pl.BlockSpec((1, B, E2), lambda t: (t, 0, 0)),
            pl.BlockSpec((1, B, E2), lambda t: (L - 1 - t, 0, 0)),
            const((H, 4 * H)), const((H, 4 * H)), const((1, 4 * H)),
            const((H, 4 * H)), const((H, 4 * H)), const((1, 4 * H)),
        ],
        out_specs=[
            pl.BlockSpec((1, B, H), lambda t: (t, 0, 0)),
            pl.BlockSpec((1, B, H), lambda t: (L - 1 - t, 0, 0)),
        ],
        out_shape=[
            jax.ShapeDtypeStruct((L, B, H), jnp.float32),
            jax.ShapeDtypeStruct((L, B, H), jnp.float32),
        ],
        scratch_shapes=[pltpu.VMEM((B, H), jnp.float32)] * 4,
        compiler_params=pltpu.CompilerParams(
            dimension_semantics=("arbitrary",),
        ),
        interpret=interpret,
    )(xw, xw, wih_f_T, whh_f_T, b_f, wih_b_T, whh_b_T, b_b)


def _attn_body(L, hsf_ref, hsb_ref, hnf_ref, hnb_ref, wof, wob, bo,
               out_ref, m, d, accf, accb):
    t = pl.program_id(0)

    @pl.when(t == 0)
    def _():
        m[...] = jnp.full_like(m, -jnp.inf)
        d[...] = jnp.zeros_like(d)
        accf[...] = jnp.zeros_like(accf)
        accb[...] = jnp.zeros_like(accb)

    hf = hsf_ref[0]
    hb = hsb_ref[0]
    s = (jnp.sum(hf * hnf_ref[0], axis=-1, keepdims=True)
         + jnp.sum(hb * hnb_ref[0], axis=-1, keepdims=True))
    m_new = jnp.maximum(m[...], s)
    alpha = jnp.exp(m[...] - m_new)
    p = jnp.exp(s - m_new)
    d[...] = d[...] * alpha + p
    accf[...] = accf[...] * alpha + p * hf
    accb[...] = accb[...] * alpha + p * hb
    m[...] = m_new

    @pl.when(t == L - 1)
    def _():
        inv = 1.0 / d[...]
        ctxf = accf[...] * inv
        ctxb = accb[...] * inv
        out_ref[...] = (
            jnp.dot(ctxf, wof[...], preferred_element_type=jnp.float32)
            + jnp.dot(ctxb, wob[...], preferred_element_type=jnp.float32)
            + bo[...]
        )


def _run_attn(hs_f, hs_b, wof, wob, bo, interpret=False):
    L, B, H = hs_f.shape
    C = wof.shape[1]
    const = lambda shape: pl.BlockSpec(shape, lambda t: (0,) * len(shape))
    return pl.pallas_call(
        functools.partial(_attn_body, L),
        grid=(L,),
        in_specs=[
            pl.BlockSpec((1, B, H), lambda t: (t, 0, 0)),
            pl.BlockSpec((1, B, H), lambda t: (t, 0, 0)),
            pl.BlockSpec((1, B, H), lambda t: (0, 0, 0)),
            pl.BlockSpec((1, B, H), lambda t: (L - 1, 0, 0)),
            const((H, C)), const((H, C)), const((1, C)),
        ],
        out_specs=pl.BlockSpec((B, C), lambda t: (0, 0)),
        out_shape=jax.ShapeDtypeStruct((B, C), jnp.float32),
        scratch_shapes=[
            pltpu.VMEM((B, 1), jnp.float32),
            pltpu.VMEM((B, 1), jnp.float32),
            pltpu.VMEM((B, H), jnp.float32),
            pltpu.VMEM((B, H), jnp.float32),
        ],
        compiler_params=pltpu.CompilerParams(
            dimension_semantics=("arbitrary",),
        ),
        interpret=interpret,
    )(hs_f, hs_b, hs_b, hs_f, wof, wob, bo)


def kernel(batch, emb, Wih_f, Whh_f, bih_f, bhh_f, Wih_b, Whh_b, bih_b, bhh_b,
           W_out, b_out):
    B, L = batch.shape
    V, E = emb.shape
    H = Whh_f.shape[1]
    C = W_out.shape[0]

    idx = batch.astype(jnp.int32).T.reshape(-1)
    emb2 = _run_pad_transpose(emb.T)
    x_wide = _make_sc_gather(V, 2 * E, L * B)(emb2, idx)
    xw = x_wide.reshape(L, B, 2 * E)

    b_f = (bih_f + bhh_f).reshape(1, 4 * H)
    b_b = (bih_b + bhh_b).reshape(1, 4 * H)
    hs_f, hs_b = _run_lstm(xw, Wih_f.T, Whh_f.T, b_f, Wih_b.T, Whh_b.T, b_b)

    woutT = W_out.T
    out = _run_attn(hs_f, hs_b, woutT[:H], woutT[H:], b_out.reshape(1, C))
    return out


# speedup vs baseline: 2.0866x; 1.4490x over previous
import functools

import jax
import jax.numpy as jnp
from jax import lax
from jax.experimental import pallas as pl
from jax.experimental.pallas import tpu as pltpu
from jax.experimental.pallas import tpu_sc as plsc


_CB = 4096


def _pack_transpose_body(E, CB, x_ref, il_ref, ir_ref, out_ref):
    f32 = jnp.float32
    te = jax.lax.dot_general(x_ref[:, 0:CB // 2], il_ref[...],
                             (((0,), (0,)), ((), ())),
                             preferred_element_type=f32)
    to = jax.lax.dot_general(x_ref[:, CB // 2:CB], ir_ref[...],
                             (((0,), (0,)), ((), ())),
                             preferred_element_type=f32)
    out_ref[...] = te + to


def _run_pad_transpose(embT):
    E, V = embT.shape
    CB = _CB
    nb = (V + CB - 1) // CB
    zeroE = jnp.zeros((E, E), jnp.float32)
    il = jnp.concatenate([jnp.eye(E, dtype=jnp.float32), zeroE], axis=1)
    ir = jnp.concatenate([zeroE, jnp.eye(E, dtype=jnp.float32)], axis=1)
    return pl.pallas_call(
        functools.partial(_pack_transpose_body, E, CB),
        grid=(nb,),
        in_specs=[
            pl.BlockSpec((E, CB), lambda j: (0, j)),
            pl.BlockSpec((E, 2 * E), lambda j: (0, 0)),
            pl.BlockSpec((E, 2 * E), lambda j: (0, 0)),
        ],
        out_specs=pl.BlockSpec((CB // 2, 2 * E), lambda j: (j, 0)),
        out_shape=jax.ShapeDtypeStruct((nb * CB // 2, 2 * E), jnp.float32),
        compiler_params=pltpu.CompilerParams(
            dimension_semantics=("arbitrary",),
        ),
    )(embT, il, ir)


def _make_sc_gather(V2, D2, N):
    info = plsc.get_sparse_core_info()
    NC, NS = info.num_cores, info.num_subcores
    NW = NC * NS
    assert N % NW == 0
    n_per_w = N // NW
    CHUNK = 400
    assert n_per_w % CHUNK == 0
    n_chunks = n_per_w // CHUNK

    mesh = plsc.VectorSubcoreMesh(core_axis_name="c", subcore_axis_name="s")

    @functools.partial(
        pl.kernel,
        out_type=jax.ShapeDtypeStruct((N, D2), jnp.float32),
        mesh=mesh,
        scratch_types=[
            pltpu.VMEM((n_per_w,), jnp.int32),
            pltpu.VMEM((CHUNK, D2), jnp.float32),
            pltpu.VMEM((CHUNK, D2), jnp.float32),
            pltpu.SemaphoreType.DMA,
            pltpu.SemaphoreType.DMA,
        ],
    )
    def gather(table_hbm, idx_hbm, out_hbm, idx_v, rows_a, rows_b, sem_a, sem_b):
        wid = lax.axis_index("s") * NC + lax.axis_index("c")
        base = wid * n_per_w
        pltpu.sync_copy(idx_hbm.at[pl.ds(base, n_per_w)], idx_v)
        bufs = ((rows_a, sem_a), (rows_b, sem_b))
        copies = []
        for c in range(n_chunks):
            rows_v, sem = bufs[c % 2]
            if c >= 2:
                copies[c - 2].wait()
                pltpu.sync_copy(rows_v, out_hbm.at[pl.ds(base + (c - 2) * CHUNK, CHUNK)])
            copies.append(pltpu.async_copy(
                table_hbm.at[idx_v.at[pl.ds(c * CHUNK, CHUNK)]], rows_v, sem))
        for c in range(n_chunks - 2, n_chunks):
            rows_v, sem = bufs[c % 2]
            copies[c].wait()
            pltpu.sync_copy(rows_v, out_hbm.at[pl.ds(base + c * CHUNK, CHUNK)])

    return gather


def _mm(a, b):
    return jax.lax.dot_general(a, b, (((1,), (0,)), ((), ())),
                               preferred_element_type=jnp.float32)


def _lstm_body(H, TL, xf_ref, xb_ref, pf_ref, pb_ref, ipad, wih_f, whh_f, b_f,
               wih_b, whh_b, b_b, hsf_ref, hsb_ref, hf, cf, hb, cb):
    j = pl.program_id(0)

    @pl.when(j == 0)
    def _():
        hf[...] = jnp.zeros_like(hf)
        cf[...] = jnp.zeros_like(cf)
        hb[...] = jnp.zeros_like(hb)
        cb[...] = jnp.zeros_like(cb)

    ip = ipad[...]
    B = hf.shape[1]
    E2 = ip.shape[0]

    def prep(x_ref, p_ref, w_ih):
        xw2 = x_ref[...].reshape(TL * B, E2)
        XT = jax.lax.dot_general(ip, xw2, (((1,), (1,)), ((), ())),
                                 preferred_element_type=jnp.float32)
        par = p_ref[...].reshape(1, TL * B)
        xsel = jnp.where(par > 0.5, XT[H:2 * H], XT[0:H])
        return _mm(w_ih[...], xsel)

    gx_f = prep(xf_ref, pf_ref, wih_f)
    gx_b = prep(xb_ref, pb_ref, wih_b)

    def substep(gx, w_hh, b, h, c):
        gates = gx + _mm(w_hh[...], h) + b[...]
        i = jax.nn.sigmoid(gates[0 * H:1 * H])
        f = jax.nn.sigmoid(gates[1 * H:2 * H])
        g = jnp.tanh(gates[2 * H:3 * H])
        o = jax.nn.sigmoid(gates[3 * H:4 * H])
        c_new = f * c + i * g
        h_new = o * jnp.tanh(c_new)
        return h_new, c_new

    h_f, c_f = hf[...], cf[...]
    h_b, c_b = hb[...], cb[...]
    for k in range(TL):
        h_f, c_f = substep(gx_f[:, k * B:(k + 1) * B], whh_f, b_f, h_f, c_f)
        hsf_ref[k] = h_f
        h_b, c_b = substep(gx_b[:, (TL - 1 - k) * B:(TL - k) * B],
                           whh_b, b_b, h_b, c_b)
        hsb_ref[TL - 1 - k] = h_b
    hf[...], cf[...] = h_f, c_f
    hb[...], cb[...] = h_b, c_b


def _run_lstm(xw, par, wih_f, whh_f, b_f, wih_b, whh_b, b_b, interpret=False):
    L, B, E2 = xw.shape
    H = whh_f.shape[1]
    TL = 8
    NB = L // TL
    const = lambda shape: pl.BlockSpec(shape, lambda j: (0,) * len(shape))
    return pl.pallas_call(
        functools.partial(_lstm_body, H, TL),
        grid=(NB,),
        in_specs=[
            pl.BlockSpec((TL, B, E2), lambda j: (j, 0, 0)),
            pl.BlockSpec((TL, B, E2), lambda j: (NB - 1 - j, 0, 0)),
            pl.BlockSpec((TL, 1, B), lambda j: (j, 0, 0)),
            pl.BlockSpec((TL, 1, B), lambda j: (NB - 1 - j, 0, 0)),
            const((E2, E2)),
            const((4 * H, H)), const((4 * H, H)), const((4 * H, 1)),
            const((4 * H, H)), const((4 * H, H)), const((4 * H, 1)),
        ],
        out_specs=[
            pl.BlockSpec((TL, H, B), lambda j: (j, 0, 0)),
            pl.BlockSpec((TL, H, B), lambda j: (NB - 1 - j, 0, 0)),
        ],
        out_shape=[
            jax.ShapeDtypeStruct((L, H, B), jnp.float32),
            jax.ShapeDtypeStruct((L, H, B), jnp.float32),
        ],
        scratch_shapes=[pltpu.VMEM((H, B), jnp.float32)] * 4,
        compiler_params=pltpu.CompilerParams(
            dimension_semantics=("arbitrary",),
        ),
        interpret=interpret,
    )(xw, xw, par, par, jnp.eye(E2, dtype=jnp.float32),
      wih_f, whh_f, b_f, wih_b, whh_b, b_b)


def _attn_body(NB, TA, hsf_ref, hsb_ref, hnf_ref, hnb_ref, wout, bo,
               out_ref, m_s, d_s, accf_s, accb_s):
    j = pl.program_id(0)

    @pl.when(j == 0)
    def _():
        m_s[...] = jnp.full_like(m_s, -jnp.inf)
        d_s[...] = jnp.zeros_like(d_s)
        accf_s[...] = jnp.zeros_like(accf_s)
        accb_s[...] = jnp.zeros_like(accb_s)

    hnf = hnf_ref[0]
    hnb = hnb_ref[0]
    m, d = m_s[...], d_s[...]
    accf, accb = accf_s[...], accb_s[...]
    for k in range(TA):
        hf = hsf_ref[k]
        hb = hsb_ref[k]
        s = (jnp.sum(hf * hnf, axis=0, keepdims=True)
             + jnp.sum(hb * hnb, axis=0, keepdims=True))
        m_new = jnp.maximum(m, s)
        alpha = jnp.exp(m - m_new)
        p = jnp.exp(s - m_new)
        d = d * alpha + p
        accf = accf * alpha + p * hf
        accb = accb * alpha + p * hb
        m = m_new
    m_s[...], d_s[...] = m, d
    accf_s[...], accb_s[...] = accf, accb

    @pl.when(j == NB - 1)
    def _():
        inv = 1.0 / d
        ctx = jnp.concatenate([accf * inv, accb * inv], axis=0)
        out_ref[...] = jax.lax.dot_general(
            ctx, wout[...], (((0,), (1,)), ((), ())),
            preferred_element_type=jnp.float32) + bo[...]


def _run_attn(hsf, hsb, wout, bo, interpret=False):
    L, H, B = hsf.shape
    C = wout.shape[0]
    TA = 8
    NB = L // TA
    const = lambda shape: pl.BlockSpec(shape, lambda j: (0,) * len(shape))
    return pl.pallas_call(
        functools.partial(_attn_body, NB, TA),
        grid=(NB,),
        in_specs=[
            pl.BlockSpec((TA, H, B), lambda j: (j, 0, 0)),
            pl.BlockSpec((TA, H, B), lambda j: (j, 0, 0)),
            pl.BlockSpec((1, H, B), lambda j: (0, 0, 0)),
            pl.BlockSpec((1, H, B), lambda j: (L - 1, 0, 0)),
            const((C, 2 * H)), const((1, C)),
        ],
        out_specs=pl.BlockSpec((B, C), lambda j: (0, 0)),
        out_shape=jax.ShapeDtypeStruct((B, C), jnp.float32),
        scratch_shapes=[
            pltpu.VMEM((1, B), jnp.float32),
            pltpu.VMEM((1, B), jnp.float32),
            pltpu.VMEM((H, B), jnp.float32),
            pltpu.VMEM((H, B), jnp.float32),
        ],
        compiler_params=pltpu.CompilerParams(
            dimension_semantics=("arbitrary",),
        ),
        interpret=interpret,
    )(hsf, hsb, hsb, hsf, wout, bo)


def kernel(batch, emb, Wih_f, Whh_f, bih_f, bhh_f, Wih_b, Whh_b, bih_b, bhh_b,
           W_out, b_out):
    B, L = batch.shape
    V, E = emb.shape
    H = Whh_f.shape[1]
    C = W_out.shape[0]

    idx = batch.astype(jnp.int32).T.reshape(-1)
    emb2 = _run_pad_transpose(emb.T)
    half = _CB // 2
    idx_row = ((idx // _CB) * half) + (idx % half)
    x_wide = _make_sc_gather(emb2.shape[0], 2 * E, L * B)(emb2, idx_row)
    xw = x_wide.reshape(L, B, 2 * E)
    par = ((idx % _CB) // half).astype(jnp.float32).reshape(L, 1, B)

    b_f = (bih_f + bhh_f).reshape(4 * H, 1)
    b_b = (bih_b + bhh_b).reshape(4 * H, 1)
    hsf, hsb = _run_lstm(xw, par, Wih_f, Whh_f, b_f, Wih_b, Whh_b, b_b)
    return _run_attn(hsf, hsb, W_out, b_out.reshape(1, C))


# speedup vs baseline: 2.2558x; 1.0811x over previous
import functools

import jax
import jax.numpy as jnp
from jax import lax
from jax.experimental import pallas as pl
from jax.experimental.pallas import tpu as pltpu
from jax.experimental.pallas import tpu_sc as plsc


_CB = 4096


def _pack_transpose_body(E, CB, x_ref, il_ref, ir_ref, out_ref):
    f32 = jnp.float32
    te = jax.lax.dot_general(x_ref[:, 0:CB // 2], il_ref[...],
                             (((0,), (0,)), ((), ())),
                             preferred_element_type=f32)
    to = jax.lax.dot_general(x_ref[:, CB // 2:CB], ir_ref[...],
                             (((0,), (0,)), ((), ())),
                             preferred_element_type=f32)
    out_ref[...] = te + to


def _run_pad_transpose(embT):
    E, V = embT.shape
    CB = _CB
    nb = (V + CB - 1) // CB
    zeroE = jnp.zeros((E, E), jnp.float32)
    il = jnp.concatenate([jnp.eye(E, dtype=jnp.float32), zeroE], axis=1)
    ir = jnp.concatenate([zeroE, jnp.eye(E, dtype=jnp.float32)], axis=1)
    return pl.pallas_call(
        functools.partial(_pack_transpose_body, E, CB),
        grid=(nb,),
        in_specs=[
            pl.BlockSpec((E, CB), lambda j: (0, j)),
            pl.BlockSpec((E, 2 * E), lambda j: (0, 0)),
            pl.BlockSpec((E, 2 * E), lambda j: (0, 0)),
        ],
        out_specs=pl.BlockSpec((CB // 2, 2 * E), lambda j: (j, 0)),
        out_shape=jax.ShapeDtypeStruct((nb * CB // 2, 2 * E), jnp.float32),
        compiler_params=pltpu.CompilerParams(
            dimension_semantics=("arbitrary",),
        ),
    )(embT, il, ir)


def _make_sc_gather(V2, D2, N):
    info = plsc.get_sparse_core_info()
    NC, NS = info.num_cores, info.num_subcores
    NW = NC * NS
    assert N % NW == 0
    n_per_w = N // NW
    CHUNK = 400
    assert n_per_w % CHUNK == 0
    n_chunks = n_per_w // CHUNK

    mesh = plsc.VectorSubcoreMesh(core_axis_name="c", subcore_axis_name="s")

    @functools.partial(
        pl.kernel,
        out_type=jax.ShapeDtypeStruct((N, D2), jnp.float32),
        mesh=mesh,
        scratch_types=[
            pltpu.VMEM((n_per_w,), jnp.int32),
            pltpu.VMEM((CHUNK, D2), jnp.float32),
            pltpu.VMEM((CHUNK, D2), jnp.float32),
            pltpu.SemaphoreType.DMA,
            pltpu.SemaphoreType.DMA,
        ],
    )
    def gather(table_hbm, idx_hbm, out_hbm, idx_v, rows_a, rows_b, sem_a, sem_b):
        wid = lax.axis_index("s") * NC + lax.axis_index("c")
        base = wid * n_per_w
        pltpu.sync_copy(idx_hbm.at[pl.ds(base, n_per_w)], idx_v)
        bufs = ((rows_a, sem_a), (rows_b, sem_b))
        copies = []
        for c in range(n_chunks):
            rows_v, sem = bufs[c % 2]
            if c >= 2:
                copies[c - 2].wait()
                pltpu.sync_copy(rows_v, out_hbm.at[pl.ds(base + (c - 2) * CHUNK, CHUNK)])
            copies.append(pltpu.async_copy(
                table_hbm.at[idx_v.at[pl.ds(c * CHUNK, CHUNK)]], rows_v, sem))
        for c in range(n_chunks - 2, n_chunks):
            rows_v, sem = bufs[c % 2]
            copies[c].wait()
            pltpu.sync_copy(rows_v, out_hbm.at[pl.ds(base + c * CHUNK, CHUNK)])

    return gather


def _mm(a, b):
    return jax.lax.dot_general(a, b, (((1,), (0,)), ((), ())),
                               preferred_element_type=jnp.float32)


def _lstm_body(H, TL, xf_ref, xb_ref, pf_ref, pb_ref, ipad, wih_f, whh_f, b_f,
               wih_b, whh_b, b_b, hsf_ref, hsb_ref, hf, cf, hb, cb):
    j = pl.program_id(0)

    @pl.when(j == 0)
    def _():
        hf[...] = jnp.zeros_like(hf)
        cf[...] = jnp.zeros_like(cf)
        hb[...] = jnp.zeros_like(hb)
        cb[...] = jnp.zeros_like(cb)

    ip = ipad[...]

    def substep(xw, par, w_ih, w_hh, b, h, c):
        xTw = jax.lax.dot_general(ip, xw, (((1,), (1,)), ((), ())),
                                  preferred_element_type=jnp.float32)
        xT = jnp.where(par > 0.5, xTw[H:2 * H], xTw[0:H])
        gates = _mm(w_ih[...], xT) + _mm(w_hh[...], h) + b[...]
        sig = lambda z: 0.5 * jnp.tanh(0.5 * z) + 0.5
        i = sig(gates[0 * H:1 * H])
        f = sig(gates[1 * H:2 * H])
        g = jnp.tanh(gates[2 * H:3 * H])
        o = sig(gates[3 * H:4 * H])
        c_new = f * c + i * g
        h_new = o * jnp.tanh(c_new)
        return h_new, c_new

    h_f, c_f = hf[...], cf[...]
    h_b, c_b = hb[...], cb[...]
    for k in range(TL):
        h_f, c_f = substep(xf_ref[k], pf_ref[k], wih_f, whh_f, b_f, h_f, c_f)
        hsf_ref[k] = h_f
        h_b, c_b = substep(xb_ref[TL - 1 - k], pb_ref[TL - 1 - k],
                           wih_b, whh_b, b_b, h_b, c_b)
        hsb_ref[TL - 1 - k] = h_b
    hf[...], cf[...] = h_f, c_f
    hb[...], cb[...] = h_b, c_b


def _run_lstm(xw, par, wih_f, whh_f, b_f, wih_b, whh_b, b_b, interpret=False):
    L, B, E2 = xw.shape
    H = whh_f.shape[1]
    TL = 8
    NB = L // TL
    const = lambda shape: pl.BlockSpec(shape, lambda j: (0,) * len(shape))
    return pl.pallas_call(
        functools.partial(_lstm_body, H, TL),
        grid=(NB,),
        in_specs=[
            pl.BlockSpec((TL, B, E2), lambda j: (j, 0, 0)),
            pl.BlockSpec((TL, B, E2), lambda j: (NB - 1 - j, 0, 0)),
            pl.BlockSpec((TL, 1, B), lambda j: (j, 0, 0)),
            pl.BlockSpec((TL, 1, B), lambda j: (NB - 1 - j, 0, 0)),
            const((E2, E2)),
            const((4 * H, H)), const((4 * H, H)), const((4 * H, 1)),
            const((4 * H, H)), const((4 * H, H)), const((4 * H, 1)),
        ],
        out_specs=[
            pl.BlockSpec((TL, H, B), lambda j: (j, 0, 0)),
            pl.BlockSpec((TL, H, B), lambda j: (NB - 1 - j, 0, 0)),
        ],
        out_shape=[
            jax.ShapeDtypeStruct((L, H, B), jnp.float32),
            jax.ShapeDtypeStruct((L, H, B), jnp.float32),
        ],
        scratch_shapes=[pltpu.VMEM((H, B), jnp.float32)] * 4,
        compiler_params=pltpu.CompilerParams(
            dimension_semantics=("arbitrary",),
        ),
        interpret=interpret,
    )(xw, xw, par, par, jnp.eye(E2, dtype=jnp.float32),
      wih_f, whh_f, b_f, wih_b, whh_b, b_b)


def _attn_body(NB, TA, hsf_ref, hsb_ref, hnf_ref, hnb_ref, wout, bo,
               out_ref, m_s, d_s, accf_s, accb_s):
    j = pl.program_id(0)

    @pl.when(j == 0)
    def _():
        m_s[...] = jnp.full_like(m_s, -jnp.inf)
        d_s[...] = jnp.zeros_like(d_s)
        accf_s[...] = jnp.zeros_like(accf_s)
        accb_s[...] = jnp.zeros_like(accb_s)

    hnf = hnf_ref[0]
    hnb = hnb_ref[0]
    m, d = m_s[...], d_s[...]
    accf, accb = accf_s[...], accb_s[...]
    for k in range(TA):
        hf = hsf_ref[k]
        hb = hsb_ref[k]
        s = (jnp.sum(hf * hnf, axis=0, keepdims=True)
             + jnp.sum(hb * hnb, axis=0, keepdims=True))
        m_new = jnp.maximum(m, s)
        alpha = jnp.exp(m - m_new)
        p = jnp.exp(s - m_new)
        d = d * alpha + p
        accf = accf * alpha + p * hf
        accb = accb * alpha + p * hb
        m = m_new
    m_s[...], d_s[...] = m, d
    accf_s[...], accb_s[...] = accf, accb

    @pl.when(j == NB - 1)
    def _():
        inv = 1.0 / d
        ctx = jnp.concatenate([accf * inv, accb * inv], axis=0)
        out_ref[...] = jax.lax.dot_general(
            ctx, wout[...], (((0,), (1,)), ((), ())),
            preferred_element_type=jnp.float32) + bo[...]


def _run_attn(hsf, hsb, wout, bo, interpret=False):
    L, H, B = hsf.shape
    C = wout.shape[0]
    TA = 8
    NB = L // TA
    const = lambda shape: pl.BlockSpec(shape, lambda j: (0,) * len(shape))
    return pl.pallas_call(
        functools.partial(_attn_body, NB, TA),
        grid=(NB,),
        in_specs=[
            pl.BlockSpec((TA, H, B), lambda j: (j, 0, 0)),
            pl.BlockSpec((TA, H, B), lambda j: (j, 0, 0)),
            pl.BlockSpec((1, H, B), lambda j: (0, 0, 0)),
            pl.BlockSpec((1, H, B), lambda j: (L - 1, 0, 0)),
            const((C, 2 * H)), const((1, C)),
        ],
        out_specs=pl.BlockSpec((B, C), lambda j: (0, 0)),
        out_shape=jax.ShapeDtypeStruct((B, C), jnp.float32),
        scratch_shapes=[
            pltpu.VMEM((1, B), jnp.float32),
            pltpu.VMEM((1, B), jnp.float32),
            pltpu.VMEM((H, B), jnp.float32),
            pltpu.VMEM((H, B), jnp.float32),
        ],
        compiler_params=pltpu.CompilerParams(
            dimension_semantics=("arbitrary",),
        ),
        interpret=interpret,
    )(hsf, hsb, hsb, hsf, wout, bo)


def kernel(batch, emb, Wih_f, Whh_f, bih_f, bhh_f, Wih_b, Whh_b, bih_b, bhh_b,
           W_out, b_out):
    B, L = batch.shape
    V, E = emb.shape
    H = Whh_f.shape[1]
    C = W_out.shape[0]

    idx = batch.astype(jnp.int32).T.reshape(-1)
    emb2 = _run_pad_transpose(emb.T)
    half = _CB // 2
    idx_row = ((idx // _CB) * half) + (idx % half)
    x_wide = _make_sc_gather(emb2.shape[0], 2 * E, L * B)(emb2, idx_row)
    xw = x_wide.reshape(L, B, 2 * E)
    par = ((idx % _CB) // half).astype(jnp.float32).reshape(L, 1, B)

    b_f = (bih_f + bhh_f).reshape(4 * H, 1)
    b_b = (bih_b + bhh_b).reshape(4 * H, 1)
    hsf, hsb = _run_lstm(xw, par, Wih_f, Whh_f, b_f, Wih_b, Whh_b, b_b)
    return _run_attn(hsf, hsb, W_out, b_out.reshape(1, C))
